# baseline (speedup 1.0000x reference)
import functools

import jax
import jax.numpy as jnp
from jax import lax
from jax.experimental import pallas as pl
from jax.experimental.pallas import tpu as pltpu
from jax.experimental.pallas import tpu_sc as plsc

_LANES = 16
_NC = 2
_NS = 16
_NW = _NC * _NS
_CHUNK = 128


@functools.lru_cache(maxsize=None)
def _build_sc_kernel(B, D):
    b_per_w = B // _NW
    n_chunks = b_per_w // _CHUNK
    groups_per_chunk = _CHUNK // _LANES
    n_sub = D // _LANES

    mesh = plsc.VectorSubcoreMesh(core_axis_name="c", subcore_axis_name="s")

    @functools.partial(
        pl.kernel,
        out_type=jax.ShapeDtypeStruct((B,), jnp.float32),
        mesh=mesh,
        compiler_params=pltpu.CompilerParams(needs_layout_passes=False,
                                             use_tc_tiling_on_sc=False),
        scratch_types=[
            pltpu.VMEM((n_chunks, _CHUNK), jnp.int32),
            pltpu.VMEM((n_chunks, _CHUNK), jnp.int32),
            pltpu.VMEM((n_chunks, _CHUNK, D), jnp.float32),
            pltpu.VMEM((n_chunks, _CHUNK, D), jnp.float32),
            pltpu.VMEM((n_chunks, _CHUNK), jnp.float32),
            pltpu.VMEM((n_chunks, _CHUNK), jnp.float32),
            pltpu.VMEM((b_per_w,), jnp.float32),
            pltpu.VMEM((_LANES,), jnp.float32),
            pltpu.SemaphoreType.DMA,
        ],
    )
    def sc_kernel(p1_hbm, p2_hbm, table_hbm, pbias_hbm, bias_hbm, out_hbm,
                  idx1_v, idx2_v, rows1_v, rows2_v, b1_v, b2_v, out_v,
                  bias_v, sem):
        wid = lax.axis_index("s") * _NC + lax.axis_index("c")
        base = wid * b_per_w

        pltpu.sync_copy(bias_hbm, bias_v)
        for c in range(n_chunks):
            pltpu.sync_copy(p1_hbm.at[pl.ds(base + c * _CHUNK, _CHUNK)],
                            idx1_v.at[c])
            pltpu.sync_copy(p2_hbm.at[pl.ds(base + c * _CHUNK, _CHUNK)],
                            idx2_v.at[c])

        copies = []
        for c in range(n_chunks):
            copies.append(pltpu.async_copy(table_hbm.at[idx1_v.at[c]],
                                           rows1_v.at[c], sem))
            copies.append(pltpu.async_copy(table_hbm.at[idx2_v.at[c]],
                                           rows2_v.at[c], sem))
            copies.append(pltpu.async_copy(pbias_hbm.at[idx1_v.at[c]],
                                           b1_v.at[c], sem))
            copies.append(pltpu.async_copy(pbias_hbm.at[idx2_v.at[c]],
                                           b2_v.at[c], sem))
        for cp in copies:
            cp.wait()

        for c in range(n_chunks):
            def group_body(g, carry, c=c):
                iota = lax.iota(jnp.int32, _LANES)
                bias_splat = bias_v[...]
                r0 = g * _LANES
                acc = jnp.zeros((_LANES,), jnp.float32)
                for j in range(_LANES):
                    r = r0 + j
                    t = None
                    for q in range(n_sub):
                        a = rows1_v[c, r, pl.ds(q * _LANES, _LANES)]
                        b = rows2_v[c, r, pl.ds(q * _LANES, _LANES)]
                        t = a * b if t is None else t + a * b
                    acc = jnp.where(iota == j, jnp.sum(t), acc)
                vb1 = b1_v[c, pl.ds(r0, _LANES)]
                vb2 = b2_v[c, pl.ds(r0, _LANES)]
                x = acc + vb1 + vb2 + bias_splat
                out_v[pl.ds(c * _CHUNK + r0, _LANES)] = (
                    1.0 / (1.0 + jnp.exp(-x)))
                return carry
            lax.fori_loop(0, groups_per_chunk, group_body, 0)

        pltpu.sync_copy(out_v, out_hbm.at[pl.ds(base, b_per_w)])

    return sc_kernel


def kernel(product1, product2, product_embedding, product_bias, bias):
    sc_kernel = _build_sc_kernel(product1.shape[0],
                                 product_embedding.shape[1])
    pbias_flat = jnp.reshape(product_bias, (-1,))
    bias_vec = jnp.broadcast_to(bias, (_LANES,)).astype(jnp.float32)
    return sc_kernel(product1.astype(jnp.int32), product2.astype(jnp.int32),
                     product_embedding, pbias_flat, bias_vec)
